# Initial kernel scaffold; baseline (speedup 1.0000x reference)
import jax
import jax.numpy as jnp
from jax.experimental import pallas as pl


def kernel(x, edge_index, edge_attr, batch, atom_emb, bond_emb, W1, b1, W2, b2, bn_gamma, bn_beta, mlp_W, mlp_b):
    raise NotImplementedError("write your pallas kernel here")


# speedup vs baseline: 4.1012x; 4.1012x over previous
import functools

import jax
import jax.numpy as jnp
from jax import lax
from jax.experimental import pallas as pl
from jax.experimental.pallas import tpu as pltpu
from jax.experimental.pallas import tpu_sc as plsc

_N = 10000
_NPAD = 10240
_E = 320000
_H = 128
_G = 64
_CH = 128
_NCHUNK_E = _E // _CH
_NTILES = 32
_LANE = 16

_mesh = plsc.VectorSubcoreMesh(core_axis_name="c", subcore_axis_name="s")


@functools.partial(pl.kernel,
                   out_type=jax.ShapeDtypeStruct((_NPAD, _H), jnp.float32),
                   mesh=_mesh,
                   scratch_types=[
                       pltpu.VMEM((_CH,), jnp.int32),
                       pltpu.VMEM((_CH, _H), jnp.float32),
                       pltpu.VMEM((_CH, _H), jnp.float32),
                   ])
def _sc_atom_encoder(xT_hbm, tab_hbm, h0_hbm, idx_v, rows_v, acc_v):
    c = lax.axis_index("c")
    s = lax.axis_index("s")
    wid = s * 2 + c
    nchunks_total = _NPAD // _CH
    nch = (nchunks_total // _NTILES) + jnp.where(
        wid < (nchunks_total % _NTILES), 1, 0)

    @pl.loop(0, nch)
    def _chunk(t):
        base = (wid + _NTILES * t) * _CH
        pltpu.sync_copy(xT_hbm.at[pl.ds(base, _CH)], idx_v)
        pltpu.sync_copy(tab_hbm.at[idx_v], acc_v)
        for f in range(1, 9):
            pltpu.sync_copy(xT_hbm.at[pl.ds(f * _NPAD + base, _CH)], idx_v)

            @pl.loop(0, _CH, step=_LANE)
            def _off(i):
                idx_v[pl.ds(i, _LANE)] = idx_v[pl.ds(i, _LANE)] + f * _H

            pltpu.sync_copy(tab_hbm.at[idx_v], rows_v)

            @pl.loop(0, _CH)
            def _acc(r):
                for j in range(0, _H, _LANE):
                    acc_v[r, pl.ds(j, _LANE)] = (
                        acc_v[r, pl.ds(j, _LANE)] + rows_v[r, pl.ds(j, _LANE)])

        pltpu.sync_copy(acc_v, h0_hbm.at[pl.ds(base, _CH)])


@functools.partial(pl.kernel,
                   out_type=jax.ShapeDtypeStruct((2, _N, _H), jnp.float32),
                   mesh=_mesh,
                   scratch_types=[
                       pltpu.VMEM((_CH,), jnp.int32),
                       pltpu.VMEM((_CH,), jnp.int32),
                       pltpu.VMEM((_CH,), jnp.int32),
                       pltpu.VMEM((_CH,), jnp.int32),
                       pltpu.VMEM((_CH,), jnp.int32),
                       pltpu.VMEM((_CH,), jnp.int32),
                       pltpu.VMEM((3 * 8 * _H,), jnp.float32),
                       pltpu.VMEM((_CH, _H), jnp.float32),
                       pltpu.VMEM((_CH, _H), jnp.float32),
                       pltpu.VMEM_SHARED((_N, _H), jnp.float32),
                       pltpu.VMEM_SHARED((512, _H), jnp.float32),
                   ])
def _sc_edge_stage(h_hbm, ei_hbm, ea_hbm, be_hbm, out_hbm,
                   src_v, dst_v, a0, a1, a2, code_v, bflat,
                   hrows, erows, agg_sh, etab_sh):
    c = lax.axis_index("c")
    s = lax.axis_index("s")
    wid = s * 2 + c

    pltpu.sync_copy(be_hbm, bflat)
    for rl in range(32):
        r = s * 32 + rl
        c0 = r // 64
        c1 = (r // 8) % 8
        c2 = r % 8
        for j in range(0, _H, _LANE):
            erows[rl, pl.ds(j, _LANE)] = (
                bflat[pl.ds(c0 * _H + j, _LANE)]
                + bflat[pl.ds(8 * _H + c1 * _H + j, _LANE)]
                + bflat[pl.ds(16 * _H + c2 * _H + j, _LANE)])
    pltpu.sync_copy(erows.at[pl.ds(0, 32)], etab_sh.at[pl.ds(s * 32, 32)])

    @pl.loop(0, _CH)
    def _zrow(r):
        for j in range(0, _H, _LANE):
            hrows[r, pl.ds(j, _LANE)] = jnp.zeros((_LANE,), jnp.float32)

    nblk = 4 + jnp.where(s < 14, 1, 0)

    @pl.loop(0, nblk)
    def _zblk(k):
        rb = (s + 16 * k) * _CH
        pltpu.sync_copy(hrows, agg_sh.at[pl.ds(rb, _CH)])

    @pl.when(s == 0)
    def _ztail():
        pltpu.sync_copy(hrows.at[pl.ds(0, 16)], agg_sh.at[pl.ds(78 * _CH, 16)])

    plsc.subcore_barrier()

    nch = (_NCHUNK_E // _NTILES) + jnp.where(
        wid < (_NCHUNK_E % _NTILES), 1, 0)

    @pl.loop(0, nch)
    def _chunk(t):
        base = (wid + _NTILES * t) * _CH
        pltpu.sync_copy(ei_hbm.at[pl.ds(base, _CH)], src_v)
        pltpu.sync_copy(ei_hbm.at[pl.ds(_E + base, _CH)], dst_v)
        pltpu.sync_copy(ea_hbm.at[pl.ds(base, _CH)], a0)
        pltpu.sync_copy(ea_hbm.at[pl.ds(_E + base, _CH)], a1)
        pltpu.sync_copy(ea_hbm.at[pl.ds(2 * _E + base, _CH)], a2)

        @pl.loop(0, _CH, step=_LANE)
        def _code(i):
            code_v[pl.ds(i, _LANE)] = (a0[pl.ds(i, _LANE)] * 64
                                       + a1[pl.ds(i, _LANE)] * 8
                                       + a2[pl.ds(i, _LANE)])

        pltpu.sync_copy(h_hbm.at[src_v], hrows)
        pltpu.sync_copy(etab_sh.at[code_v], erows)

        @pl.loop(0, _CH)
        def _relu(r):
            for j in range(0, _H, _LANE):
                hv = hrows[r, pl.ds(j, _LANE)]
                ev = erows[r, pl.ds(j, _LANE)]
                hrows[r, pl.ds(j, _LANE)] = jnp.maximum(hv + ev, 0.0)

        pltpu.sync_copy(hrows, agg_sh.at[dst_v], add=True)

    plsc.subcore_barrier()

    @pl.loop(0, nblk)
    def _dblk(k):
        rb = (s + 16 * k) * _CH
        pltpu.sync_copy(agg_sh.at[pl.ds(rb, _CH)], out_hbm.at[c, pl.ds(rb, _CH)])

    @pl.when(s == 0)
    def _dtail():
        pltpu.sync_copy(agg_sh.at[pl.ds(78 * _CH, 16)],
                        out_hbm.at[c, pl.ds(78 * _CH, 16)])


def _tc_dense_layer(h, p0, p1, w1, b1, w2, b2, gamma, beta):
    def body(h_ref, p0_ref, p1_ref, w1_ref, b1_ref, w2_ref, b2_ref,
             g_ref, be_ref, o_ref):
        a = h_ref[...] + p0_ref[...] + p1_ref[...]
        t = jnp.dot(a, w1_ref[...], preferred_element_type=jnp.float32)
        t = jnp.maximum(t + b1_ref[...], 0.0)
        u = jnp.dot(t, w2_ref[...], preferred_element_type=jnp.float32)
        u = u + b2_ref[...]
        mu = jnp.mean(u, axis=0, keepdims=True)
        var = jnp.mean((u - mu) * (u - mu), axis=0, keepdims=True)
        v = (u - mu) * lax.rsqrt(var + 1e-5) * g_ref[...] + be_ref[...]
        o_ref[...] = h_ref[...] + jnp.maximum(v, 0.0)

    return pl.pallas_call(
        body,
        out_shape=jax.ShapeDtypeStruct((_N, _H), jnp.float32),
    )(h, p0, p1, w1, b1, w2, b2, gamma, beta)


def _tc_pool_mlp(h, batchT, mlp_w, mlp_b):
    def body(h_ref, b_ref, w_ref, bias_ref, o_ref):
        gid = lax.broadcasted_iota(jnp.int32, (_G, _N), 0)
        oh = (b_ref[...] == gid).astype(jnp.float32)
        sums = jnp.dot(oh, h_ref[...], preferred_element_type=jnp.float32)
        counts = jnp.sum(oh, axis=1)
        pooled = sums / jnp.maximum(counts, 1.0)[:, None]
        o_ref[...] = (jnp.dot(pooled, w_ref[...],
                              preferred_element_type=jnp.float32)
                      + bias_ref[...])

    return pl.pallas_call(
        body,
        out_shape=jax.ShapeDtypeStruct((_G, _H), jnp.float32),
    )(h, batchT, mlp_w, mlp_b)


def kernel(x, edge_index, edge_attr, batch, atom_emb, bond_emb,
           W1, b1, W2, b2, bn_gamma, bn_beta, mlp_W, mlp_b):
    xT = jnp.pad(x.astype(jnp.int32).T, ((0, 0), (0, _NPAD - _N))).reshape(-1)
    tab = atom_emb.reshape(9 * 128, _H)
    ei = edge_index.astype(jnp.int32).reshape(-1)
    eaT = edge_attr.astype(jnp.int32).T.reshape(-1)
    beflat = bond_emb.reshape(-1)
    batchT = jnp.broadcast_to(batch.astype(jnp.int32)[None, :], (_G, _N))

    h = _sc_atom_encoder(xT, tab)[:_N]
    for i in range(3):
        p = _sc_edge_stage(h, ei, eaT, beflat)
        h = _tc_dense_layer(h, p[0], p[1], W1[i], b1[i].reshape(1, _H),
                            W2[i], b2[i].reshape(1, _H),
                            bn_gamma[i].reshape(1, _H),
                            bn_beta[i].reshape(1, _H))
    return _tc_pool_mlp(h, batchT, mlp_W, mlp_b.reshape(1, _H))
